# baseline (speedup 1.0000x reference)
import jax
import jax.numpy as jnp
from jax import lax
from jax.experimental import pallas as pl
from jax.experimental.pallas import tpu as pltpu

NUM_ROWS = 128
ROW_LEN = 100000
BLOCK_COLS = 2048
NUM_BLOCKS = ROW_LEN // BLOCK_COLS
TAIL_COL0 = NUM_BLOCKS * BLOCK_COLS
NBUF = 4
STRIPES = 2
STRIPE_ROWS = NUM_ROWS // STRIPES


def _scan_block(t, colg, bv, bi):
    bmax = jnp.max(t, axis=1, keepdims=True)
    cand = jnp.where(t == bmax, colg, ROW_LEN)
    barg = jnp.min(cand, axis=1, keepdims=True)
    better = bmax > bv
    return jnp.where(better, bmax, bv), jnp.where(better, barg, bi)


def _body(x_hbm, xt_ref, out_ref, *scratch):
    bufs = scratch[:NBUF]
    sems = scratch[NBUF:NBUF + NBUF * STRIPES]

    def start(b):
        buf = bufs[b % NBUF]
        for r in range(STRIPES):
            pltpu.make_async_copy(
                x_hbm.at[pl.ds(r * STRIPE_ROWS, STRIPE_ROWS),
                         pl.ds(b * BLOCK_COLS, BLOCK_COLS)],
                buf.at[pl.ds(r * STRIPE_ROWS, STRIPE_ROWS), :],
                sems[(b % NBUF) * STRIPES + r],
            ).start()

    def wait(b):
        buf = bufs[b % NBUF]
        for r in range(STRIPES):
            pltpu.make_async_copy(
                x_hbm.at[pl.ds(r * STRIPE_ROWS, STRIPE_ROWS),
                         pl.ds(b * BLOCK_COLS, BLOCK_COLS)],
                buf.at[pl.ds(r * STRIPE_ROWS, STRIPE_ROWS), :],
                sems[(b % NBUF) * STRIPES + r],
            ).wait()

    for b in range(NBUF):
        start(b)

    col = lax.broadcasted_iota(jnp.int32, (NUM_ROWS, BLOCK_COLS), 1)
    bv = jnp.full((NUM_ROWS, 1), -jnp.inf, jnp.float32)
    bi = jnp.zeros((NUM_ROWS, 1), jnp.int32)

    for b in range(NUM_BLOCKS):
        wait(b)
        t = bufs[b % NBUF][...]
        if b + NBUF < NUM_BLOCKS:
            pass
        bv, bi = _scan_block(t, col + b * BLOCK_COLS, bv, bi)
        if b + NBUF < NUM_BLOCKS:
            start(b + NBUF)

    colg = col + TAIL_COL0
    t = jnp.where(colg < ROW_LEN, xt_ref[...], -jnp.inf)
    bv, bi = _scan_block(t, colg, bv, bi)

    out_ref[...] = bi


@jax.jit
def _argmax_impl(logits):
    out = pl.pallas_call(
        _body,
        grid=(1,),
        in_specs=[
            pl.BlockSpec(memory_space=pltpu.MemorySpace.HBM),
            pl.BlockSpec((NUM_ROWS, BLOCK_COLS), lambda j: (0, NUM_BLOCKS)),
        ],
        out_specs=pl.BlockSpec((NUM_ROWS, 1), lambda j: (0, 0)),
        out_shape=jax.ShapeDtypeStruct((NUM_ROWS, 1), jnp.int32),
        scratch_shapes=(
            [pltpu.VMEM((NUM_ROWS, BLOCK_COLS), jnp.float32)
             for _ in range(NBUF)]
            + [pltpu.SemaphoreType.DMA] * (NBUF * STRIPES)
        ),
    )(logits, logits)
    return out.reshape(NUM_ROWS)


def kernel(logits, temperatures):
    return _argmax_impl(logits)


# speedup vs baseline: 1.0086x; 1.0086x over previous
import jax
import jax.numpy as jnp
from jax import lax
from jax.experimental import pallas as pl
from jax.experimental.pallas import tpu as pltpu

NUM_ROWS = 128
ROW_LEN = 100000
BLOCK_ROWS = 32
NUM_BLOCKS = NUM_ROWS // BLOCK_ROWS
NBUF = 3
SPLITS = 4
SPLIT_ROWS = BLOCK_ROWS // SPLITS


def _body(x_hbm, out_ref, *scratch):
    bufs = scratch[:NBUF]
    sems = scratch[NBUF:]

    def copies(b):
        buf = bufs[b % NBUF]
        for r in range(SPLITS):
            yield pltpu.make_async_copy(
                x_hbm.at[pl.ds(b * BLOCK_ROWS + r * SPLIT_ROWS, SPLIT_ROWS), :],
                buf.at[pl.ds(r * SPLIT_ROWS, SPLIT_ROWS), :],
                sems[(b % NBUF) * SPLITS + r],
            )

    for b in range(NBUF):
        for cp in copies(b):
            cp.start()

    col = lax.broadcasted_iota(jnp.int32, (BLOCK_ROWS, ROW_LEN), 1)
    for b in range(NUM_BLOCKS):
        for cp in copies(b):
            cp.wait()
        t = bufs[b % NBUF][...]
        bmax = jnp.max(t, axis=1, keepdims=True)
        cand = jnp.where(t == bmax, col, ROW_LEN)
        barg = jnp.min(cand, axis=1, keepdims=True)
        out_ref[pl.ds(b * BLOCK_ROWS, BLOCK_ROWS), :] = barg
        if b + NBUF < NUM_BLOCKS:
            for cp in copies(b + NBUF):
                cp.start()


@jax.jit
def _argmax_impl(logits):
    out = pl.pallas_call(
        _body,
        grid=(1,),
        in_specs=[pl.BlockSpec(memory_space=pltpu.MemorySpace.HBM)],
        out_specs=pl.BlockSpec((NUM_ROWS, 1), lambda j: (0, 0)),
        out_shape=jax.ShapeDtypeStruct((NUM_ROWS, 1), jnp.int32),
        scratch_shapes=(
            [pltpu.VMEM((BLOCK_ROWS, ROW_LEN), jnp.float32)
             for _ in range(NBUF)]
            + [pltpu.SemaphoreType.DMA] * (NBUF * SPLITS)
        ),
    )(logits)
    return out.reshape(NUM_ROWS)


def kernel(logits, temperatures):
    return _argmax_impl(logits)
